# baseline (speedup 1.0000x reference)
import functools

import jax
import jax.numpy as jnp
from jax import lax
from jax.experimental import pallas as pl
from jax.experimental.pallas import tpu as pltpu
from jax.experimental.pallas import tpu_sc as plsc

B, S, H = 2, 2048, 1024
E, K = 16, 2
FE, FS = 512, 1408
T = B * S
N = T * K
M = 256
NT = N // M + E
PN = NT * M

NC, NS = 2, 16
NW = NC * NS


def _pack_rows(a):
    ab = a.astype(jnp.bfloat16).astype(jnp.float32)
    lo = lax.bitcast_convert_type(ab[:, : H // 2], jnp.uint32)
    hi = lax.bitcast_convert_type(ab[:, H // 2 :], jnp.uint32)
    return lax.bitcast_convert_type(hi | (lo >> 16), jnp.float32)


def _unpack_rows(p):
    u = lax.bitcast_convert_type(p, jnp.uint32)
    lo = lax.bitcast_convert_type(u << 16, jnp.float32)
    hi = lax.bitcast_convert_type(u & jnp.uint32(0xFFFF0000), jnp.float32)
    return jnp.concatenate([lo, hi], axis=1)


def _router_body(x_ref, wg_ref, oi_ref, ow_ref, ox_ref):
    logits = lax.dot_general(x_ref[...], wg_ref[...], (((1,), (0,)), ((), ())),
                             preferred_element_type=jnp.float32)
    lane = lax.broadcasted_iota(jnp.int32, logits.shape, 1)
    logits = jnp.where(lane < E, logits, -1e30)
    m1 = jnp.max(logits, axis=1, keepdims=True)
    i1 = jnp.min(jnp.where(logits == m1, lane, 127), axis=1, keepdims=True)
    l2 = jnp.where(lane == i1, -1e30, logits)
    m2 = jnp.max(l2, axis=1, keepdims=True)
    i2 = jnp.min(jnp.where(l2 == m2, lane, 127), axis=1, keepdims=True)
    e2 = jnp.exp(m2 - m1)
    wa = 1.0 / (1.0 + e2)
    wb = 1.0 - wa
    oi_ref[...] = jnp.where(lane == 0, i1, jnp.where(lane == 1, i2, 0))
    ow_ref[...] = jnp.where(lane == 0, wa, jnp.where(lane == 1, wb, 0.0))
    ox_ref[...] = _pack_rows(x_ref[...])


def _router(xf, wgp):
    bt = 512
    return pl.pallas_call(
        _router_body,
        grid=(T // bt,),
        in_specs=[
            pl.BlockSpec((bt, H), lambda i: (i, 0)),
            pl.BlockSpec((H, 128), lambda i: (0, 0)),
        ],
        out_specs=[
            pl.BlockSpec((bt, 128), lambda i: (i, 0)),
            pl.BlockSpec((bt, 128), lambda i: (i, 0)),
            pl.BlockSpec((bt, H // 2), lambda i: (i, 0)),
        ],
        out_shape=[
            jax.ShapeDtypeStruct((T, 128), jnp.int32),
            jax.ShapeDtypeStruct((T, 128), jnp.float32),
            jax.ShapeDtypeStruct((T, H // 2), jnp.float32),
        ],
    )(xf, wgp)


_SC_MESH = plsc.VectorSubcoreMesh(core_axis_name="c", subcore_axis_name="s",
                                  num_cores=NC, num_subcores=NS)
_D_PW = T // NW
_D_CH = 32
_D_NCH = _D_PW // _D_CH


@functools.partial(
    pl.kernel, mesh=_SC_MESH,
    out_type=jax.ShapeDtypeStruct((PN, H // 2), jnp.float32),
    scratch_types=[
        pltpu.VMEM((_D_NCH, _D_CH), jnp.int32),
        pltpu.VMEM((_D_NCH, _D_CH), jnp.int32),
        pltpu.VMEM((_D_CH, H // 2), jnp.float32),
        pltpu.VMEM((_D_CH, H // 2), jnp.float32),
        pltpu.SemaphoreType.DMA,
        pltpu.SemaphoreType.DMA,
        pltpu.SemaphoreType.DMA,
        pltpu.SemaphoreType.DMA,
    ],
)
def _dispatch(xfb_hbm, de_hbm, do_hbm, xg_hbm, ie_v, io_v, rows0, rows1,
              lsem0, lsem1, ssem0, ssem1):
    wid = lax.axis_index("s") * NC + lax.axis_index("c")
    base = wid * _D_PW
    pltpu.sync_copy(de_hbm.at[wid], ie_v)
    pltpu.sync_copy(do_hbm.at[wid], io_v)
    bufs = (rows0, rows1)
    lsems = (lsem0, lsem1)
    ssems = (ssem0, ssem1)

    def load(c, b):
        return pltpu.async_copy(
            xfb_hbm.at[pl.ds(base + c * _D_CH, _D_CH)], bufs[b], lsems[b])

    stores = [None, None]
    ls = [load(0, 0), None]
    for c in range(_D_NCH):
        b = c & 1
        nb = 1 - b
        if c + 1 < _D_NCH:
            if stores[nb] is not None:
                stores[nb][0].wait()
                stores[nb][1].wait()
            ls[nb] = load(c + 1, nb)
        ls[b].wait()
        se = pltpu.async_copy(bufs[b], xg_hbm.at[ie_v.at[c]], ssems[b])
        so = pltpu.async_copy(bufs[b], xg_hbm.at[io_v.at[c]], ssems[b])
        stores[b] = (se, so)
    for st in stores:
        if st is not None:
            st[0].wait()
            st[1].wait()


_G_PW = N // NW
_G_CH = 32
_G_NCH = _G_PW // _G_CH


@functools.partial(
    pl.kernel, mesh=_SC_MESH,
    out_type=jax.ShapeDtypeStruct((N, H // 2), jnp.float32),
    scratch_types=[
        pltpu.VMEM((_G_PW,), jnp.int32),
        pltpu.VMEM((_G_CH, H // 2), jnp.float32),
        pltpu.VMEM((_G_CH, H // 2), jnp.float32),
        pltpu.SemaphoreType.DMA,
        pltpu.SemaphoreType.DMA,
        pltpu.SemaphoreType.DMA,
        pltpu.SemaphoreType.DMA,
    ],
)
def _combine_gather(yg_hbm, idx_hbm, out_hbm, idx_v, rows0, rows1,
                    gsem0, gsem1, ssem0, ssem1):
    wid = lax.axis_index("s") * NC + lax.axis_index("c")
    base = wid * _G_PW
    pltpu.sync_copy(idx_hbm.at[pl.ds(base, _G_PW)], idx_v)
    bufs = (rows0, rows1)
    gsems = (gsem0, gsem1)
    ssems = (ssem0, ssem1)

    def gather(c, b):
        return pltpu.async_copy(
            yg_hbm.at[idx_v.at[pl.ds(c * _G_CH, _G_CH)]], bufs[b], gsems[b])

    stores = [None, None]
    gs = [gather(0, 0), None]
    for c in range(_G_NCH):
        b = c & 1
        nb = 1 - b
        if c + 1 < _G_NCH:
            if stores[nb] is not None:
                stores[nb].wait()
            gs[nb] = gather(c + 1, nb)
        gs[b].wait()
        stores[b] = pltpu.async_copy(
            bufs[b], out_hbm.at[pl.ds(base + c * _G_CH, _G_CH)], ssems[b])
    for st in stores:
        if st is not None:
            st.wait()


def _ffn_body(te_ref, x_ref, w1_ref, w3_ref, w2_ref, o_ref):
    del te_ref
    x = _unpack_rows(x_ref[...]).astype(jnp.bfloat16)
    h1 = lax.dot_general(x, w1_ref[0], (((1,), (1,)), ((), ())),
                         preferred_element_type=jnp.float32)
    h3 = lax.dot_general(x, w3_ref[0], (((1,), (1,)), ((), ())),
                         preferred_element_type=jnp.float32)
    hh = (h1 * lax.logistic(h1) * h3).astype(jnp.bfloat16)
    y = lax.dot_general(hh, w2_ref[0], (((1,), (1,)), ((), ())),
                        preferred_element_type=jnp.float32)
    o_ref[...] = _pack_rows(y)


def _ffn(tile_expert, xg, w1b, w3b, w2b):
    grid_spec = pltpu.PrefetchScalarGridSpec(
        num_scalar_prefetch=1,
        grid=(NT,),
        in_specs=[
            pl.BlockSpec((M, H // 2), lambda i, te: (i, 0)),
            pl.BlockSpec((1, FE, H), lambda i, te: (te[i], 0, 0)),
            pl.BlockSpec((1, FE, H), lambda i, te: (te[i], 0, 0)),
            pl.BlockSpec((1, H, FE), lambda i, te: (te[i], 0, 0)),
        ],
        out_specs=pl.BlockSpec((M, H // 2), lambda i, te: (i, 0)),
    )
    return pl.pallas_call(
        _ffn_body,
        grid_spec=grid_spec,
        out_shape=jax.ShapeDtypeStruct((PN, H // 2), jnp.float32),
    )(tile_expert, xg, w1b, w3b, w2b)


def _shared_body(x_ref, w1_ref, w3_ref, w2_ref, o_ref):
    x = x_ref[...].astype(jnp.bfloat16)
    h1 = lax.dot_general(x, w1_ref[...], (((1,), (1,)), ((), ())),
                         preferred_element_type=jnp.float32)
    h3 = lax.dot_general(x, w3_ref[...], (((1,), (1,)), ((), ())),
                         preferred_element_type=jnp.float32)
    hh = (h1 * lax.logistic(h1) * h3).astype(jnp.bfloat16)
    o_ref[...] = lax.dot_general(hh, w2_ref[...], (((1,), (1,)), ((), ())),
                                 preferred_element_type=jnp.float32)


def _shared(xf, ws1b, ws3b, ws2b):
    bt = 512
    return pl.pallas_call(
        _shared_body,
        grid=(T // bt,),
        in_specs=[
            pl.BlockSpec((bt, H), lambda i: (i, 0)),
            pl.BlockSpec((FS, H), lambda i: (0, 0)),
            pl.BlockSpec((FS, H), lambda i: (0, 0)),
            pl.BlockSpec((H, FS), lambda i: (0, 0)),
        ],
        out_specs=pl.BlockSpec((bt, H), lambda i: (i, 0)),
        out_shape=jax.ShapeDtypeStruct((T, H), jnp.float32),
    )(xf, ws1b, ws3b, ws2b)


def _add_body(s_ref, w_ref, y0_ref, y1_ref, o_ref):
    w0 = w_ref[:, 0:1]
    w1 = w_ref[:, 1:2]
    y0 = _unpack_rows(y0_ref[0])
    y1 = _unpack_rows(y1_ref[0])
    o_ref[...] = s_ref[...] + w0 * y0 + w1 * y1


def _final_add(shared, topw_p, yt2):
    bt = 512
    return pl.pallas_call(
        _add_body,
        grid=(T // bt,),
        in_specs=[
            pl.BlockSpec((bt, H), lambda i: (i, 0)),
            pl.BlockSpec((bt, 128), lambda i: (i, 0)),
            pl.BlockSpec((1, bt, H // 2), lambda i: (0, i, 0)),
            pl.BlockSpec((1, bt, H // 2), lambda i: (1, i, 0)),
        ],
        out_specs=pl.BlockSpec((bt, H), lambda i: (i, 0)),
        out_shape=jax.ShapeDtypeStruct((T, H), jnp.float32),
    )(shared, topw_p, yt2, yt2)


def kernel(x, Wg, W1, W2, W3, Ws1, Ws2, Ws3):
    xf = x.reshape(T, H)
    wgp = jnp.zeros((H, 128), jnp.float32).at[:, :E].set(Wg.T)
    topi_p, topw_p, xfp = _router(xf, wgp)
    topi = topi_p[:, :K]

    flat_e = topi.reshape(-1)
    oh = (flat_e[:, None] == jnp.arange(E)[None, :]).astype(jnp.int32)
    rank = jnp.take_along_axis(jnp.cumsum(oh, axis=0), flat_e[:, None], 1)[:, 0] - 1
    counts = oh.sum(axis=0)
    padded = ((counts + M - 1) // M) * M
    pstart = jnp.concatenate([jnp.zeros(1, padded.dtype), jnp.cumsum(padded)])[:E]
    dest = (pstart[flat_e] + rank).astype(jnp.int32)
    tile_expert = (jnp.sum(jnp.arange(NT)[:, None] * M >= pstart[None, :], axis=1)
                   - 1).astype(jnp.int32)
    de3 = dest[0::K].reshape(NW, _D_NCH, _D_CH)
    do3 = dest[1::K].reshape(NW, _D_NCH, _D_CH)
    cidx = jnp.concatenate([dest[0::K], dest[1::K]])

    xg = _dispatch(xfp, de3, do3)
    yg = _ffn(tile_expert, xg, W1.astype(jnp.bfloat16),
              W3.astype(jnp.bfloat16), W2.astype(jnp.bfloat16))
    shared = _shared(xf, Ws1.astype(jnp.bfloat16), Ws3.astype(jnp.bfloat16),
                     Ws2.astype(jnp.bfloat16))
    yt2 = _combine_gather(yg, cidx).reshape(2, T, H // 2)
    out = _final_add(shared, topw_p, yt2)
    return out.reshape(B, S, H)


# speedup vs baseline: 1.1755x; 1.1755x over previous
import functools

import jax
import jax.numpy as jnp
from jax import lax
from jax.experimental import pallas as pl
from jax.experimental.pallas import tpu as pltpu
from jax.experimental.pallas import tpu_sc as plsc

B, S, H = 2, 2048, 1024
E, K = 16, 2
FE, FS = 512, 1408
T = B * S
N = T * K
M = 256
NT = N // M + E
PN = NT * M

NC, NS = 2, 16
NW = NC * NS


def _pack_rows(a):
    ab = a.astype(jnp.bfloat16).astype(jnp.float32)
    lo = lax.bitcast_convert_type(ab[:, : H // 2], jnp.uint32)
    hi = lax.bitcast_convert_type(ab[:, H // 2 :], jnp.uint32)
    return lax.bitcast_convert_type(hi | (lo >> 16), jnp.float32)


def _unpack_rows(p):
    u = lax.bitcast_convert_type(p, jnp.uint32)
    lo = lax.bitcast_convert_type(u << 16, jnp.float32)
    hi = lax.bitcast_convert_type(u & jnp.uint32(0xFFFF0000), jnp.float32)
    return jnp.concatenate([lo, hi], axis=1)


def _router_body(x_ref, wg_ref, oi_ref, ow_ref, ox_ref):
    logits = lax.dot_general(x_ref[...], wg_ref[...], (((1,), (0,)), ((), ())),
                             preferred_element_type=jnp.float32)
    lane = lax.broadcasted_iota(jnp.int32, logits.shape, 1)
    logits = jnp.where(lane < E, logits, -1e30)
    m1 = jnp.max(logits, axis=1, keepdims=True)
    i1 = jnp.min(jnp.where(logits == m1, lane, 127), axis=1, keepdims=True)
    l2 = jnp.where(lane == i1, -1e30, logits)
    m2 = jnp.max(l2, axis=1, keepdims=True)
    i2 = jnp.min(jnp.where(l2 == m2, lane, 127), axis=1, keepdims=True)
    e2 = jnp.exp(m2 - m1)
    wa = 1.0 / (1.0 + e2)
    wb = 1.0 - wa
    oi_ref[...] = jnp.where(lane == 0, i1, jnp.where(lane == 1, i2, 0))
    ow_ref[...] = jnp.where(lane == 0, wa, jnp.where(lane == 1, wb, 0.0))
    ox_ref[...] = _pack_rows(x_ref[...])


def _router(xf, wgp):
    bt = 512
    return pl.pallas_call(
        _router_body,
        grid=(T // bt,),
        in_specs=[
            pl.BlockSpec((bt, H), lambda i: (i, 0)),
            pl.BlockSpec((H, 128), lambda i: (0, 0)),
        ],
        out_specs=[
            pl.BlockSpec((bt, 128), lambda i: (i, 0)),
            pl.BlockSpec((bt, 128), lambda i: (i, 0)),
            pl.BlockSpec((bt, H // 2), lambda i: (i, 0)),
        ],
        out_shape=[
            jax.ShapeDtypeStruct((T, 128), jnp.int32),
            jax.ShapeDtypeStruct((T, 128), jnp.float32),
            jax.ShapeDtypeStruct((T, H // 2), jnp.float32),
        ],
    )(xf, wgp)


_SC_MESH = plsc.VectorSubcoreMesh(core_axis_name="c", subcore_axis_name="s",
                                  num_cores=NC, num_subcores=NS)
_D_PW = T // NW
_D_CH = 32
_D_NCH = _D_PW // _D_CH


@functools.partial(
    pl.kernel, mesh=_SC_MESH,
    out_type=jax.ShapeDtypeStruct((PN, H // 2), jnp.float32),
    scratch_types=[
        pltpu.VMEM((_D_NCH, _D_CH), jnp.int32),
        pltpu.VMEM((_D_NCH, _D_CH), jnp.int32),
        pltpu.VMEM((_D_CH, H // 2), jnp.float32),
        pltpu.VMEM((_D_CH, H // 2), jnp.float32),
        pltpu.SemaphoreType.DMA,
        pltpu.SemaphoreType.DMA,
        pltpu.SemaphoreType.DMA,
        pltpu.SemaphoreType.DMA,
    ],
)
def _dispatch(xfb_hbm, de_hbm, do_hbm, xg_hbm, ie_v, io_v, rows0, rows1,
              lsem0, lsem1, ssem0, ssem1):
    wid = lax.axis_index("s") * NC + lax.axis_index("c")
    base = wid * _D_PW
    pltpu.sync_copy(de_hbm.at[wid], ie_v)
    pltpu.sync_copy(do_hbm.at[wid], io_v)
    bufs = (rows0, rows1)
    lsems = (lsem0, lsem1)
    ssems = (ssem0, ssem1)

    def load(c, b):
        return pltpu.async_copy(
            xfb_hbm.at[pl.ds(base + c * _D_CH, _D_CH)], bufs[b], lsems[b])

    stores = [None, None]
    ls = [load(0, 0), None]
    for c in range(_D_NCH):
        b = c & 1
        nb = 1 - b
        if c + 1 < _D_NCH:
            if stores[nb] is not None:
                stores[nb][0].wait()
                stores[nb][1].wait()
            ls[nb] = load(c + 1, nb)
        ls[b].wait()
        se = pltpu.async_copy(bufs[b], xg_hbm.at[ie_v.at[c]], ssems[b])
        so = pltpu.async_copy(bufs[b], xg_hbm.at[io_v.at[c]], ssems[b])
        stores[b] = (se, so)
    for st in stores:
        if st is not None:
            st[0].wait()
            st[1].wait()


_G_PW = N // NW
_G_CH = 32
_G_NCH = _G_PW // _G_CH


@functools.partial(
    pl.kernel, mesh=_SC_MESH,
    out_type=jax.ShapeDtypeStruct((N, H // 2), jnp.float32),
    scratch_types=[
        pltpu.VMEM((_G_PW,), jnp.int32),
        pltpu.VMEM((_G_CH, H // 2), jnp.float32),
        pltpu.VMEM((_G_CH, H // 2), jnp.float32),
        pltpu.SemaphoreType.DMA,
        pltpu.SemaphoreType.DMA,
        pltpu.SemaphoreType.DMA,
        pltpu.SemaphoreType.DMA,
    ],
)
def _combine_gather(yg_hbm, idx_hbm, out_hbm, idx_v, rows0, rows1,
                    gsem0, gsem1, ssem0, ssem1):
    wid = lax.axis_index("s") * NC + lax.axis_index("c")
    base = wid * _G_PW
    pltpu.sync_copy(idx_hbm.at[pl.ds(base, _G_PW)], idx_v)
    bufs = (rows0, rows1)
    gsems = (gsem0, gsem1)
    ssems = (ssem0, ssem1)

    def gather(c, b):
        return pltpu.async_copy(
            yg_hbm.at[idx_v.at[pl.ds(c * _G_CH, _G_CH)]], bufs[b], gsems[b])

    stores = [None, None]
    gs = [gather(0, 0), None]
    for c in range(_G_NCH):
        b = c & 1
        nb = 1 - b
        if c + 1 < _G_NCH:
            if stores[nb] is not None:
                stores[nb].wait()
            gs[nb] = gather(c + 1, nb)
        gs[b].wait()
        stores[b] = pltpu.async_copy(
            bufs[b], out_hbm.at[pl.ds(base + c * _G_CH, _G_CH)], ssems[b])
    for st in stores:
        if st is not None:
            st.wait()


def _ffn_body(te_ref, x_ref, w1_ref, w3_ref, w2_ref, o_ref):
    del te_ref
    x = _unpack_rows(x_ref[...]).astype(jnp.bfloat16)
    w1 = w1_ref[0].astype(jnp.bfloat16)
    w3 = w3_ref[0].astype(jnp.bfloat16)
    w2 = w2_ref[0].astype(jnp.bfloat16)
    h1 = lax.dot_general(x, w1, (((1,), (1,)), ((), ())),
                         preferred_element_type=jnp.float32)
    h3 = lax.dot_general(x, w3, (((1,), (1,)), ((), ())),
                         preferred_element_type=jnp.float32)
    hh = (h1 * lax.logistic(h1) * h3).astype(jnp.bfloat16)
    y = lax.dot_general(hh, w2, (((1,), (1,)), ((), ())),
                        preferred_element_type=jnp.float32)
    o_ref[...] = _pack_rows(y)


def _ffn(tile_expert, xg, w1b, w3b, w2b):
    grid_spec = pltpu.PrefetchScalarGridSpec(
        num_scalar_prefetch=1,
        grid=(NT,),
        in_specs=[
            pl.BlockSpec((M, H // 2), lambda i, te: (i, 0)),
            pl.BlockSpec((1, FE, H), lambda i, te: (te[i], 0, 0)),
            pl.BlockSpec((1, FE, H), lambda i, te: (te[i], 0, 0)),
            pl.BlockSpec((1, H, FE), lambda i, te: (te[i], 0, 0)),
        ],
        out_specs=pl.BlockSpec((M, H // 2), lambda i, te: (i, 0)),
    )
    return pl.pallas_call(
        _ffn_body,
        grid_spec=grid_spec,
        out_shape=jax.ShapeDtypeStruct((PN, H // 2), jnp.float32),
    )(tile_expert, xg, w1b, w3b, w2b)


def _shared_body(x_ref, w1_ref, w3_ref, w2_ref, o_ref):
    x = x_ref[...].astype(jnp.bfloat16)
    w1 = w1_ref[...].astype(jnp.bfloat16)
    w3 = w3_ref[...].astype(jnp.bfloat16)
    w2 = w2_ref[...].astype(jnp.bfloat16)
    h1 = lax.dot_general(x, w1, (((1,), (1,)), ((), ())),
                         preferred_element_type=jnp.float32)
    h3 = lax.dot_general(x, w3, (((1,), (1,)), ((), ())),
                         preferred_element_type=jnp.float32)
    hh = (h1 * lax.logistic(h1) * h3).astype(jnp.bfloat16)
    o_ref[...] = lax.dot_general(hh, w2, (((1,), (1,)), ((), ())),
                                 preferred_element_type=jnp.float32)


def _shared(xf, ws1b, ws3b, ws2b):
    bt = 512
    return pl.pallas_call(
        _shared_body,
        grid=(T // bt,),
        in_specs=[
            pl.BlockSpec((bt, H), lambda i: (i, 0)),
            pl.BlockSpec((FS, H), lambda i: (0, 0)),
            pl.BlockSpec((FS, H), lambda i: (0, 0)),
            pl.BlockSpec((H, FS), lambda i: (0, 0)),
        ],
        out_specs=pl.BlockSpec((bt, H), lambda i: (i, 0)),
        out_shape=jax.ShapeDtypeStruct((T, H), jnp.float32),
    )(xf, ws1b, ws3b, ws2b)


def _add_body(s_ref, w_ref, y0_ref, y1_ref, o_ref):
    w0 = w_ref[:, 0:1]
    w1 = w_ref[:, 1:2]
    y0 = _unpack_rows(y0_ref[0])
    y1 = _unpack_rows(y1_ref[0])
    o_ref[...] = s_ref[...] + w0 * y0 + w1 * y1


def _final_add(shared, topw_p, yt2):
    bt = 512
    return pl.pallas_call(
        _add_body,
        grid=(T // bt,),
        in_specs=[
            pl.BlockSpec((bt, H), lambda i: (i, 0)),
            pl.BlockSpec((bt, 128), lambda i: (i, 0)),
            pl.BlockSpec((1, bt, H // 2), lambda i: (0, i, 0)),
            pl.BlockSpec((1, bt, H // 2), lambda i: (1, i, 0)),
        ],
        out_specs=pl.BlockSpec((bt, H), lambda i: (i, 0)),
        out_shape=jax.ShapeDtypeStruct((T, H), jnp.float32),
    )(shared, topw_p, yt2, yt2)


def kernel(x, Wg, W1, W2, W3, Ws1, Ws2, Ws3):
    xf = x.reshape(T, H)
    wgp = jnp.zeros((H, 128), jnp.float32).at[:, :E].set(Wg.T)
    topi_p, topw_p, xfp = _router(xf, wgp)
    topi = topi_p[:, :K]

    flat_e = topi.reshape(-1)
    oh = (flat_e[:, None] == jnp.arange(E)[None, :]).astype(jnp.int32)
    rank = jnp.take_along_axis(jnp.cumsum(oh, axis=0), flat_e[:, None], 1)[:, 0] - 1
    counts = oh.sum(axis=0)
    padded = ((counts + M - 1) // M) * M
    pstart = jnp.concatenate([jnp.zeros(1, padded.dtype), jnp.cumsum(padded)])[:E]
    dest = (pstart[flat_e] + rank).astype(jnp.int32)
    tile_expert = (jnp.sum(jnp.arange(NT)[:, None] * M >= pstart[None, :], axis=1)
                   - 1).astype(jnp.int32)
    de3 = dest[0::K].reshape(NW, _D_NCH, _D_CH)
    do3 = dest[1::K].reshape(NW, _D_NCH, _D_CH)
    cidx = jnp.concatenate([dest[0::K], dest[1::K]])

    xg = _dispatch(xfp, de3, do3)
    yg = _ffn(tile_expert, xg, W1, W3, W2)
    shared = _shared(xf, Ws1, Ws3, Ws2)
    yt2 = _combine_gather(yg, cidx).reshape(2, T, H // 2)
    out = _final_add(shared, topw_p, yt2)
    return out.reshape(B, S, H)


# speedup vs baseline: 1.2663x; 1.0773x over previous
import functools

import jax
import jax.numpy as jnp
from jax import lax
from jax.experimental import pallas as pl
from jax.experimental.pallas import tpu as pltpu
from jax.experimental.pallas import tpu_sc as plsc

B, S, H = 2, 2048, 1024
E, K = 16, 2
FE, FS = 512, 1408
T = B * S
N = T * K
M = 256
NT = N // M + E
PN = NT * M

NC, NS = 2, 16
NW = NC * NS


def _pack_rows(a):
    ab = a.astype(jnp.bfloat16).astype(jnp.float32)
    lo = lax.bitcast_convert_type(ab[:, : H // 2], jnp.uint32)
    hi = lax.bitcast_convert_type(ab[:, H // 2 :], jnp.uint32)
    return lax.bitcast_convert_type(hi | (lo >> 16), jnp.float32)


def _unpack_rows(p):
    u = lax.bitcast_convert_type(p, jnp.uint32)
    lo = lax.bitcast_convert_type(u << 16, jnp.float32)
    hi = lax.bitcast_convert_type(u & jnp.uint32(0xFFFF0000), jnp.float32)
    return jnp.concatenate([lo, hi], axis=1)


def _router_body(x_ref, wg_ref, oi_ref, ow_ref, ox_ref, or_ref, cnt_ref, offs):
    logits = lax.dot_general(x_ref[...], wg_ref[...], (((1,), (0,)), ((), ())),
                             preferred_element_type=jnp.float32)
    lane = lax.broadcasted_iota(jnp.int32, logits.shape, 1)
    logits = jnp.where(lane < E, logits, -1e30)
    m1 = jnp.max(logits, axis=1, keepdims=True)
    i1 = jnp.min(jnp.where(logits == m1, lane, 127), axis=1, keepdims=True)
    l2 = jnp.where(lane == i1, -1e30, logits)
    m2 = jnp.max(l2, axis=1, keepdims=True)
    i2 = jnp.min(jnp.where(l2 == m2, lane, 127), axis=1, keepdims=True)
    e2 = jnp.exp(m2 - m1)
    wa = 1.0 / (1.0 + e2)
    wb = 1.0 - wa
    oi_ref[...] = jnp.where(lane == 0, i1, jnp.where(lane == 1, i2, 0))
    ow_ref[...] = jnp.where(lane == 0, wa, jnp.where(lane == 1, wb, 0.0))
    ox_ref[...] = _pack_rows(x_ref[...])

    @pl.when(pl.program_id(0) == 0)
    def _():
        offs[...] = jnp.zeros_like(offs)

    bt = logits.shape[0]
    oh1 = (lane == i1).astype(jnp.float32)
    oh2 = (lane == i2).astype(jnp.float32)
    row = lax.broadcasted_iota(jnp.int32, (bt, bt), 0)
    col = lax.broadcasted_iota(jnp.int32, (bt, bt), 1)
    ltri = (col < row).astype(jnp.float32)
    c1 = lax.dot_general(ltri, oh1, (((1,), (0,)), ((), ())),
                         preferred_element_type=jnp.float32)
    c2 = lax.dot_general(ltri, oh2, (((1,), (0,)), ((), ())),
                         preferred_element_type=jnp.float32)
    base = c1 + c2 + offs[...]
    r0 = jnp.sum(jnp.where(lane == i1, base, 0.0), axis=1, keepdims=True)
    r1 = jnp.sum(jnp.where(lane == i2, base + oh1, 0.0), axis=1, keepdims=True)
    or_ref[...] = jnp.where(lane == 0, r0, jnp.where(lane == 1, r1, 0.0)
                            ).astype(jnp.int32)
    offs[...] = offs[...] + jnp.sum(oh1 + oh2, axis=0, keepdims=True)
    cnt_ref[...] = offs[...]


def _router(xf, wgp):
    bt = 512
    return pl.pallas_call(
        _router_body,
        grid=(T // bt,),
        in_specs=[
            pl.BlockSpec((bt, H), lambda i: (i, 0)),
            pl.BlockSpec((H, 128), lambda i: (0, 0)),
        ],
        out_specs=[
            pl.BlockSpec((bt, 128), lambda i: (i, 0)),
            pl.BlockSpec((bt, 128), lambda i: (i, 0)),
            pl.BlockSpec((bt, H // 2), lambda i: (i, 0)),
            pl.BlockSpec((bt, 128), lambda i: (i, 0)),
            pl.BlockSpec((1, 128), lambda i: (0, 0)),
        ],
        out_shape=[
            jax.ShapeDtypeStruct((T, 128), jnp.int32),
            jax.ShapeDtypeStruct((T, 128), jnp.float32),
            jax.ShapeDtypeStruct((T, H // 2), jnp.float32),
            jax.ShapeDtypeStruct((T, 128), jnp.int32),
            jax.ShapeDtypeStruct((1, 128), jnp.float32),
        ],
        scratch_shapes=[pltpu.VMEM((1, 128), jnp.float32)],
    )(xf, wgp)


def _plumb_body(cnt_ref, oi_ref, rk_ref, dest_ref, te_ref):
    lane1 = lax.broadcasted_iota(jnp.int32, (1, 128), 1)
    c = cnt_ref[...]
    padded = jnp.where(lane1 < E, jnp.floor((c + (M - 1)) / M) * M, 0.0)
    srow = lax.broadcasted_iota(jnp.int32, (128, 128), 0)
    scol = lax.broadcasted_iota(jnp.int32, (128, 128), 1)
    utri = (srow < scol).astype(jnp.float32)
    pstart = lax.dot_general(padded, utri, (((1,), (0,)), ((), ())),
                             preferred_element_type=jnp.float32)

    bt = oi_ref.shape[0]
    lane = lax.broadcasted_iota(jnp.int32, (bt, 128), 1)
    e0 = oi_ref[:, 0:1]
    e1 = oi_ref[:, 1:2]
    s0 = jnp.sum(jnp.where(lane == e0, pstart, 0.0), axis=1, keepdims=True)
    s1 = jnp.sum(jnp.where(lane == e1, pstart, 0.0), axis=1, keepdims=True)
    d0 = rk_ref[:, 0:1] + s0.astype(jnp.int32)
    d1 = rk_ref[:, 1:2] + s1.astype(jnp.int32)
    dest_ref[...] = jnp.where(lane == 0, d0, jnp.where(lane == 1, d1, 0))

    @pl.when(pl.program_id(0) == 0)
    def _():
        jm = (lane1 * M).astype(jnp.float32)
        acc = jnp.zeros((1, 128), jnp.float32)
        for e in range(E):
            pe = jnp.sum(jnp.where(lane1 == e, pstart, 0.0), axis=1,
                         keepdims=True)
            acc = acc + (jm >= pe).astype(jnp.float32)
        te_ref[...] = (acc - 1.0).astype(jnp.int32)


def _plumb(cnt, topi_p, rk_p):
    bt = 2048
    return pl.pallas_call(
        _plumb_body,
        grid=(T // bt,),
        in_specs=[
            pl.BlockSpec((1, 128), lambda i: (0, 0)),
            pl.BlockSpec((bt, 128), lambda i: (i, 0)),
            pl.BlockSpec((bt, 128), lambda i: (i, 0)),
        ],
        out_specs=[
            pl.BlockSpec((bt, 128), lambda i: (i, 0)),
            pl.BlockSpec((1, 128), lambda i: (0, 0)),
        ],
        out_shape=[
            jax.ShapeDtypeStruct((T, 128), jnp.int32),
            jax.ShapeDtypeStruct((1, 128), jnp.int32),
        ],
    )(cnt, topi_p, rk_p)


_SC_MESH = plsc.VectorSubcoreMesh(core_axis_name="c", subcore_axis_name="s",
                                  num_cores=NC, num_subcores=NS)
_D_PW = T // NW
_D_CH = 32
_D_NCH = _D_PW // _D_CH


@functools.partial(
    pl.kernel, mesh=_SC_MESH,
    out_type=jax.ShapeDtypeStruct((PN, H // 2), jnp.float32),
    scratch_types=[
        pltpu.VMEM((_D_NCH, _D_CH), jnp.int32),
        pltpu.VMEM((_D_NCH, _D_CH), jnp.int32),
        pltpu.VMEM((_D_CH, H // 2), jnp.float32),
        pltpu.VMEM((_D_CH, H // 2), jnp.float32),
        pltpu.SemaphoreType.DMA,
        pltpu.SemaphoreType.DMA,
        pltpu.SemaphoreType.DMA,
        pltpu.SemaphoreType.DMA,
    ],
)
def _dispatch(xfb_hbm, de_hbm, do_hbm, xg_hbm, ie_v, io_v, rows0, rows1,
              lsem0, lsem1, ssem0, ssem1):
    wid = lax.axis_index("s") * NC + lax.axis_index("c")
    base = wid * _D_PW
    pltpu.sync_copy(de_hbm.at[wid], ie_v)
    pltpu.sync_copy(do_hbm.at[wid], io_v)
    bufs = (rows0, rows1)
    lsems = (lsem0, lsem1)
    ssems = (ssem0, ssem1)

    def load(c, b):
        return pltpu.async_copy(
            xfb_hbm.at[pl.ds(base + c * _D_CH, _D_CH)], bufs[b], lsems[b])

    stores = [None, None]
    ls = [load(0, 0), None]
    for c in range(_D_NCH):
        b = c & 1
        nb = 1 - b
        if c + 1 < _D_NCH:
            if stores[nb] is not None:
                stores[nb][0].wait()
                stores[nb][1].wait()
            ls[nb] = load(c + 1, nb)
        ls[b].wait()
        se = pltpu.async_copy(bufs[b], xg_hbm.at[ie_v.at[c]], ssems[b])
        so = pltpu.async_copy(bufs[b], xg_hbm.at[io_v.at[c]], ssems[b])
        stores[b] = (se, so)
    for st in stores:
        if st is not None:
            st[0].wait()
            st[1].wait()


_G_PW = N // NW
_G_CH = 32
_G_NCH = _G_PW // _G_CH


@functools.partial(
    pl.kernel, mesh=_SC_MESH,
    out_type=jax.ShapeDtypeStruct((N, H // 2), jnp.float32),
    scratch_types=[
        pltpu.VMEM((_G_PW,), jnp.int32),
        pltpu.VMEM((_G_CH, H // 2), jnp.float32),
        pltpu.VMEM((_G_CH, H // 2), jnp.float32),
        pltpu.SemaphoreType.DMA,
        pltpu.SemaphoreType.DMA,
        pltpu.SemaphoreType.DMA,
        pltpu.SemaphoreType.DMA,
    ],
)
def _combine_gather(yg_hbm, idx_hbm, out_hbm, idx_v, rows0, rows1,
                    gsem0, gsem1, ssem0, ssem1):
    wid = lax.axis_index("s") * NC + lax.axis_index("c")
    base = wid * _G_PW
    pltpu.sync_copy(idx_hbm.at[pl.ds(base, _G_PW)], idx_v)
    bufs = (rows0, rows1)
    gsems = (gsem0, gsem1)
    ssems = (ssem0, ssem1)

    def gather(c, b):
        return pltpu.async_copy(
            yg_hbm.at[idx_v.at[pl.ds(c * _G_CH, _G_CH)]], bufs[b], gsems[b])

    stores = [None, None]
    gs = [gather(0, 0), None]
    for c in range(_G_NCH):
        b = c & 1
        nb = 1 - b
        if c + 1 < _G_NCH:
            if stores[nb] is not None:
                stores[nb].wait()
            gs[nb] = gather(c + 1, nb)
        gs[b].wait()
        stores[b] = pltpu.async_copy(
            bufs[b], out_hbm.at[pl.ds(base + c * _G_CH, _G_CH)], ssems[b])
    for st in stores:
        if st is not None:
            st.wait()


def _ffn_body(te_ref, x_ref, w1_ref, w3_ref, w2_ref, o_ref):
    del te_ref
    x = _unpack_rows(x_ref[...])
    h1 = lax.dot_general(x, w1_ref[0], (((1,), (1,)), ((), ())),
                         preferred_element_type=jnp.float32)
    h3 = lax.dot_general(x, w3_ref[0], (((1,), (1,)), ((), ())),
                         preferred_element_type=jnp.float32)
    hh = h1 * lax.logistic(h1) * h3
    y = lax.dot_general(hh, w2_ref[0], (((1,), (1,)), ((), ())),
                        preferred_element_type=jnp.float32)
    o_ref[...] = _pack_rows(y)


def _ffn(tile_expert, xg, w1b, w3b, w2b):
    grid_spec = pltpu.PrefetchScalarGridSpec(
        num_scalar_prefetch=1,
        grid=(NT,),
        in_specs=[
            pl.BlockSpec((M, H // 2), lambda i, te: (i, 0)),
            pl.BlockSpec((1, FE, H), lambda i, te: (te[i], 0, 0)),
            pl.BlockSpec((1, FE, H), lambda i, te: (te[i], 0, 0)),
            pl.BlockSpec((1, H, FE), lambda i, te: (te[i], 0, 0)),
        ],
        out_specs=pl.BlockSpec((M, H // 2), lambda i, te: (i, 0)),
    )
    return pl.pallas_call(
        _ffn_body,
        grid_spec=grid_spec,
        out_shape=jax.ShapeDtypeStruct((PN, H // 2), jnp.float32),
    )(tile_expert, xg, w1b, w3b, w2b)


def _shared_body(x_ref, w1_ref, w3_ref, w2_ref, o_ref):
    x = x_ref[...]
    h1 = lax.dot_general(x, w1_ref[...], (((1,), (1,)), ((), ())),
                         preferred_element_type=jnp.float32)
    h3 = lax.dot_general(x, w3_ref[...], (((1,), (1,)), ((), ())),
                         preferred_element_type=jnp.float32)
    hh = h1 * lax.logistic(h1) * h3
    o_ref[...] = lax.dot_general(hh, w2_ref[...], (((1,), (1,)), ((), ())),
                                 preferred_element_type=jnp.float32)


def _shared(xf, ws1b, ws3b, ws2b):
    bt = 512
    return pl.pallas_call(
        _shared_body,
        grid=(T // bt,),
        in_specs=[
            pl.BlockSpec((bt, H), lambda i: (i, 0)),
            pl.BlockSpec((FS, H), lambda i: (0, 0)),
            pl.BlockSpec((FS, H), lambda i: (0, 0)),
            pl.BlockSpec((H, FS), lambda i: (0, 0)),
        ],
        out_specs=pl.BlockSpec((bt, H), lambda i: (i, 0)),
        out_shape=jax.ShapeDtypeStruct((T, H), jnp.float32),
    )(xf, ws1b, ws3b, ws2b)


def _add_body(s_ref, w_ref, y0_ref, y1_ref, o_ref):
    w0 = w_ref[:, 0:1]
    w1 = w_ref[:, 1:2]
    y0 = _unpack_rows(y0_ref[0])
    y1 = _unpack_rows(y1_ref[0])
    o_ref[...] = s_ref[...] + w0 * y0 + w1 * y1


def _final_add(shared, topw_p, yt2):
    bt = 512
    return pl.pallas_call(
        _add_body,
        grid=(T // bt,),
        in_specs=[
            pl.BlockSpec((bt, H), lambda i: (i, 0)),
            pl.BlockSpec((bt, 128), lambda i: (i, 0)),
            pl.BlockSpec((1, bt, H // 2), lambda i: (0, i, 0)),
            pl.BlockSpec((1, bt, H // 2), lambda i: (1, i, 0)),
        ],
        out_specs=pl.BlockSpec((bt, H), lambda i: (i, 0)),
        out_shape=jax.ShapeDtypeStruct((T, H), jnp.float32),
    )(shared, topw_p, yt2, yt2)


def kernel(x, Wg, W1, W2, W3, Ws1, Ws2, Ws3):
    xf = x.reshape(T, H)
    wgp = jnp.zeros((H, 128), jnp.float32).at[:, :E].set(Wg.T)
    topi_p, topw_p, xfp, rk_p, cnt = _router(xf, wgp)
    dest_p, te_p = _plumb(cnt, topi_p, rk_p)
    tile_expert = te_p[0, :NT]
    d0 = dest_p[:, 0]
    d1 = dest_p[:, 1]
    de3 = d0.reshape(NW, _D_NCH, _D_CH)
    do3 = d1.reshape(NW, _D_NCH, _D_CH)
    cidx = jnp.concatenate([d0, d1])

    xg = _dispatch(xfp, de3, do3)
    yg = _ffn(tile_expert, xg, W1, W3, W2)
    shared = _shared(xf, Ws1, Ws3, Ws2)
    yt2 = _combine_gather(yg, cidx).reshape(2, T, H // 2)
    out = _final_add(shared, topw_p, yt2)
    return out.reshape(B, S, H)
